# baseline (speedup 1.0000x reference)
import functools

import jax
import jax.numpy as jnp
from jax import lax
from jax.experimental import pallas as pl
from jax.experimental.pallas import tpu as pltpu
from jax.experimental.pallas import tpu_sc as plsc

_CH = 64
_FB = 4
_DEPTH = 2


@functools.cache
def _make_kernel(B, D, L, NC, NS):
    NW = NC * NS
    b_per_w = B // NW
    NCH = b_per_w // _CH
    JU = D // L
    mesh = plsc.VectorSubcoreMesh(core_axis_name="c", subcore_axis_name="s")

    @functools.partial(
        pl.kernel,
        mesh=mesh,
        out_type=jax.ShapeDtypeStruct((NW, L), jnp.float32),
        scratch_types=[
            pltpu.VMEM((b_per_w,), jnp.int32),
            pltpu.VMEM((NCH, _CH, D), jnp.float32),
            pltpu.VMEM((_FB, _CH, D), jnp.float32),
            pltpu.VMEM((L,), jnp.float32),
            pltpu.SemaphoreType.DMA,
            pltpu.SemaphoreType.DMA,
            pltpu.SemaphoreType.DMA,
        ],
    )
    def k(label_hbm, feat_hbm, centers_hbm, out_hbm, idx_v, rows_v, feat_v,
          part_v, sem_g, sem_f, sem_i):
        wid = lax.axis_index("s") * NC + lax.axis_index("c")
        base = wid * b_per_w

        def start_feat(c):
            return pltpu.async_copy(
                feat_hbm.at[pl.ds(base + c * _CH, _CH)],
                feat_v.at[c % _FB], sem_f)

        def start_gather(c):
            return pltpu.async_copy(
                centers_hbm.at[idx_v.at[pl.ds(c * _CH, _CH)]],
                rows_v.at[c], sem_g)

        idxs = [pltpu.async_copy(
            label_hbm.at[pl.ds(base + c * _CH, _CH)],
            idx_v.at[pl.ds(c * _CH, _CH)], sem_i) for c in range(NCH)]
        feats = [start_feat(c) for c in range(_DEPTH)]
        gathers = []
        for c in range(_DEPTH):
            idxs[c].wait()
            gathers.append(start_gather(c))

        accs = tuple(jnp.zeros((L,), jnp.float32) for _ in range(2 * JU))
        for c in range(NCH):
            if c + _DEPTH < NCH:
                idxs[c + _DEPTH].wait()
                gathers.append(start_gather(c + _DEPTH))
                feats.append(start_feat(c + _DEPTH))
            with jax.named_scope(f"wait{c}"):
                gathers[c].wait()
                feats[c].wait()
            fbuf = c % _FB

            def row_body(i, accs, c=c, fbuf=fbuf):
                i2 = i * 2
                new = []
                for u in range(2):
                    for j in range(JU):
                        f = feat_v[fbuf, i2 + u, pl.ds(j * L, L)]
                        r = rows_v[c, i2 + u, pl.ds(j * L, L)]
                        d = f - r
                        new.append(accs[u * JU + j] + d * d)
                return tuple(new)

            with jax.named_scope(f"cmp{c}"):
                accs = lax.fori_loop(0, _CH // 2, row_body, accs)

        tot = accs[0]
        for j in range(1, 2 * JU):
            tot = tot + accs[j]
        part_v[...] = tot
        pltpu.sync_copy(part_v, out_hbm.at[wid])

    return k


def kernel(label, feat, centers):
    B, D = feat.shape
    info = plsc.get_sparse_core_info()
    k = _make_kernel(B, D, info.num_lanes, info.num_cores, info.num_subcores)
    partials = k(label, feat, centers)
    return jnp.sum(partials) / (2.0 * B)


# speedup vs baseline: 1.0027x; 1.0027x over previous
import functools

import jax
import jax.numpy as jnp
from jax import lax
from jax.experimental import pallas as pl
from jax.experimental.pallas import tpu as pltpu
from jax.experimental.pallas import tpu_sc as plsc

_CH = 64
_FB = 4
_DEPTH = 2


@functools.cache
def _make_kernel(B, D, L, NC, NS):
    NW = NC * NS
    b_per_w = B // NW
    NCH = b_per_w // _CH
    JU = D // L
    mesh = plsc.VectorSubcoreMesh(core_axis_name="c", subcore_axis_name="s")

    @functools.partial(
        pl.kernel,
        mesh=mesh,
        out_type=jax.ShapeDtypeStruct((NW, L), jnp.float32),
        scratch_types=[
            pltpu.VMEM((b_per_w,), jnp.int32),
            pltpu.VMEM((NCH, _CH, D), jnp.float32),
            pltpu.VMEM((_FB, _CH, D), jnp.float32),
            pltpu.VMEM((L,), jnp.float32),
            pltpu.SemaphoreType.DMA,
            pltpu.SemaphoreType.DMA,
            pltpu.SemaphoreType.DMA,
        ],
    )
    def k(label_hbm, feat_hbm, centers_hbm, out_hbm, idx_v, rows_v, feat_v,
          part_v, sem_g, sem_f, sem_i):
        wid = lax.axis_index("s") * NC + lax.axis_index("c")
        base = wid * b_per_w

        def start_feat(c):
            return pltpu.async_copy(
                feat_hbm.at[pl.ds(base + c * _CH, _CH)],
                feat_v.at[c % _FB], sem_f)

        def start_gather(c):
            return pltpu.async_copy(
                centers_hbm.at[idx_v.at[pl.ds(c * _CH, _CH)]],
                rows_v.at[c], sem_g)

        idxs = [pltpu.async_copy(
            label_hbm.at[pl.ds(base + c * _CH, _CH)],
            idx_v.at[pl.ds(c * _CH, _CH)], sem_i) for c in range(NCH)]
        feats = [start_feat(c) for c in range(_DEPTH)]
        gathers = []
        for c in range(_DEPTH):
            idxs[c].wait()
            gathers.append(start_gather(c))

        accs = tuple(jnp.zeros((L,), jnp.float32) for _ in range(2 * JU))
        for c in range(NCH):
            if c + _DEPTH < NCH:
                idxs[c + _DEPTH].wait()
                gathers.append(start_gather(c + _DEPTH))
                feats.append(start_feat(c + _DEPTH))
            gathers[c].wait()
            feats[c].wait()
            fbuf = c % _FB

            def row_body(i, accs, c=c, fbuf=fbuf):
                i2 = i * 2
                new = []
                for u in range(2):
                    for j in range(JU):
                        f = feat_v[fbuf, i2 + u, pl.ds(j * L, L)]
                        r = rows_v[c, i2 + u, pl.ds(j * L, L)]
                        d = f - r
                        new.append(accs[u * JU + j] + d * d)
                return tuple(new)

            accs = lax.fori_loop(0, _CH // 2, row_body, accs)

        tot = accs[0]
        for j in range(1, 2 * JU):
            tot = tot + accs[j]
        part_v[...] = tot
        pltpu.sync_copy(part_v, out_hbm.at[wid])

    return k


def kernel(label, feat, centers):
    B, D = feat.shape
    info = plsc.get_sparse_core_info()
    k = _make_kernel(B, D, info.num_lanes, info.num_cores, info.num_subcores)
    partials = k(label, feat, centers)
    return jnp.sum(partials) / (2.0 * B)


# speedup vs baseline: 1.0083x; 1.0055x over previous
import functools

import jax
import jax.numpy as jnp
from jax import lax
from jax.experimental import pallas as pl
from jax.experimental.pallas import tpu as pltpu
from jax.experimental.pallas import tpu_sc as plsc

_CHUNKS = (64, 64, 128, 128, 128)
_FB = 3


@functools.cache
def _make_kernel(B, D, L, NC, NS):
    NW = NC * NS
    b_per_w = B // NW
    assert sum(_CHUNKS) == b_per_w
    NCH = len(_CHUNKS)
    CM = max(_CHUNKS)
    JU = D // L
    offs = [sum(_CHUNKS[:c]) for c in range(NCH)]
    mesh = plsc.VectorSubcoreMesh(core_axis_name="c", subcore_axis_name="s")

    @functools.partial(
        pl.kernel,
        mesh=mesh,
        out_type=jax.ShapeDtypeStruct((NW, L), jnp.float32),
        scratch_types=[
            pltpu.VMEM((b_per_w,), jnp.int32),
            pltpu.VMEM((b_per_w, D), jnp.float32),
            pltpu.VMEM((_FB, CM, D), jnp.float32),
            pltpu.VMEM((L,), jnp.float32),
            pltpu.SemaphoreType.DMA,
            pltpu.SemaphoreType.DMA,
            pltpu.SemaphoreType.DMA,
        ],
    )
    def k(label_hbm, feat_hbm, centers_hbm, out_hbm, idx_v, rows_v, feat_v,
          part_v, sem_g, sem_f, sem_i):
        wid = lax.axis_index("s") * NC + lax.axis_index("c")
        base = wid * b_per_w

        def start_feat(c):
            return pltpu.async_copy(
                feat_hbm.at[pl.ds(base + offs[c], _CHUNKS[c])],
                feat_v.at[c % _FB, pl.ds(0, _CHUNKS[c])], sem_f)

        def start_gather(c):
            return pltpu.async_copy(
                centers_hbm.at[idx_v.at[pl.ds(offs[c], _CHUNKS[c])]],
                rows_v.at[pl.ds(offs[c], _CHUNKS[c])], sem_g)

        idxs = [pltpu.async_copy(
            label_hbm.at[pl.ds(base + offs[c], _CHUNKS[c])],
            idx_v.at[pl.ds(offs[c], _CHUNKS[c])], sem_i) for c in range(NCH)]
        feats = [start_feat(c) for c in range(_FB)]
        gathers = []
        for c in range(2):
            idxs[c].wait()
            gathers.append(start_gather(c))

        accs = tuple(jnp.zeros((L,), jnp.float32) for _ in range(2 * JU))
        for c in range(NCH):
            if c + 2 < NCH:
                idxs[c + 2].wait()
                gathers.append(start_gather(c + 2))
            gathers[c].wait()
            feats[c].wait()
            fbuf = c % _FB

            def row_body(i, accs, c=c, fbuf=fbuf):
                i2 = i * 2
                new = []
                for u in range(2):
                    for j in range(JU):
                        f = feat_v[fbuf, i2 + u, pl.ds(j * L, L)]
                        r = rows_v[offs[c] + i2 + u, pl.ds(j * L, L)]
                        d = f - r
                        new.append(accs[u * JU + j] + d * d)
                return tuple(new)

            accs = lax.fori_loop(0, _CHUNKS[c] // 2, row_body, accs)
            if c + _FB < NCH:
                feats.append(start_feat(c + _FB))

        tot = accs[0]
        for j in range(1, 2 * JU):
            tot = tot + accs[j]
        part_v[...] = tot
        pltpu.sync_copy(part_v, out_hbm.at[wid])

    return k


def kernel(label, feat, centers):
    B, D = feat.shape
    info = plsc.get_sparse_core_info()
    k = _make_kernel(B, D, info.num_lanes, info.num_cores, info.num_subcores)
    partials = k(label, feat, centers)
    return jnp.sum(partials) / (2.0 * B)
